# Initial kernel scaffold; baseline (speedup 1.0000x reference)
import jax
import jax.numpy as jnp
from jax.experimental import pallas as pl


def kernel(x, adj, W):
    raise NotImplementedError("write your pallas kernel here")


# speedup vs baseline: 1.0365x; 1.0365x over previous
import functools

import jax
import jax.numpy as jnp
from jax.experimental import pallas as pl
from jax.experimental.pallas import tpu as pltpu


def _gcn_body(x_ref, adj_ref, w_ref, out_ref, supp_ref):
    @pl.when(pl.program_id(0) == 0)
    def _():
        supp_ref[:] = jnp.dot(
            x_ref[:], w_ref[:], preferred_element_type=jnp.float32
        )

    acc = jnp.dot(adj_ref[:], supp_ref[:], preferred_element_type=jnp.float32)
    out_ref[:] = jnp.maximum(acc, 0.0)


@functools.partial(jax.jit, static_argnames=("block_m",))
def _gcn(x, adj, W, block_m):
    n, d_in = x.shape
    d_hid = W.shape[1]
    grid = (n // block_m,)
    return pl.pallas_call(
        _gcn_body,
        grid=grid,
        in_specs=[
            pl.BlockSpec((n, d_in), lambda i: (0, 0)),
            pl.BlockSpec((block_m, n), lambda i: (i, 0)),
            pl.BlockSpec((d_in, d_hid), lambda i: (0, 0)),
        ],
        out_specs=pl.BlockSpec((block_m, d_hid), lambda i: (i, 0)),
        out_shape=jax.ShapeDtypeStruct((n, d_hid), jnp.float32),
        scratch_shapes=[pltpu.VMEM((n, d_hid), jnp.float32)],
        compiler_params=pltpu.CompilerParams(
            dimension_semantics=("arbitrary",),
        ),
    )(x, adj, W)


def kernel(x, adj, W):
    n = adj.shape[0]
    for bm in (400, 200, 80, 40, 8, 1):
        if n % bm == 0:
            block_m = bm
            break
    return _gcn(x, adj, W, block_m)


# speedup vs baseline: 1.0368x; 1.0004x over previous
import functools

import jax
import jax.numpy as jnp
from jax.experimental import pallas as pl
from jax.experimental.pallas import tpu as pltpu


def _gcn_body(x_ref, adj_ref, w_ref, out_ref, supp_ref):
    @pl.when(pl.program_id(0) == 0)
    def _():
        supp_ref[:] = jnp.dot(
            x_ref[:], w_ref[:], preferred_element_type=jnp.float32
        ).astype(jnp.bfloat16)

    acc = jnp.dot(
        adj_ref[:].astype(jnp.bfloat16),
        supp_ref[:],
        preferred_element_type=jnp.float32,
    )
    out_ref[:] = jnp.maximum(acc, 0.0)


@functools.partial(jax.jit, static_argnames=("block_m",))
def _gcn(x, adj, W, block_m):
    n, d_in = x.shape
    d_hid = W.shape[1]
    grid = (n // block_m,)
    return pl.pallas_call(
        _gcn_body,
        grid=grid,
        in_specs=[
            pl.BlockSpec((n, d_in), lambda i: (0, 0)),
            pl.BlockSpec((block_m, n), lambda i: (i, 0)),
            pl.BlockSpec((d_in, d_hid), lambda i: (0, 0)),
        ],
        out_specs=pl.BlockSpec((block_m, d_hid), lambda i: (i, 0)),
        out_shape=jax.ShapeDtypeStruct((n, d_hid), jnp.float32),
        scratch_shapes=[pltpu.VMEM((n, d_hid), jnp.bfloat16)],
        compiler_params=pltpu.CompilerParams(
            dimension_semantics=("arbitrary",),
        ),
    )(x, adj, W)


def kernel(x, adj, W):
    n = adj.shape[0]
    for bm in (400, 200, 80, 40, 8, 1):
        if n % bm == 0:
            block_m = bm
            break
    return _gcn(x, adj, W, block_m)
